# Initial kernel scaffold; baseline (speedup 1.0000x reference)
import jax
import jax.numpy as jnp
from jax.experimental import pallas as pl


def kernel(x, wq1, wk1, wv1, wq2, wk2, wv2, conv1_w, conv2_w, bn2_g, bn2_b, conv5_w, bn5_g, bn5_b):
    raise NotImplementedError("write your pallas kernel here")


# speedup vs baseline: 4.9905x; 4.9905x over previous
import functools
import math

import jax
import jax.numpy as jnp
from jax.experimental import pallas as pl
from jax.experimental.pallas import tpu as pltpu

K = 20
N = 1024
N1P = 512
NV = 511


def _lrelu(t):
    return jnp.where(t >= 0, t, 0.2 * t)


def _topk_step(work, colid, src):
    mxv = jnp.max(work, axis=1, keepdims=True)
    sel = jnp.where(work == mxv, colid, jnp.int32(1 << 30))
    amin = jnp.min(sel, axis=1, keepdims=True)
    onehot = (colid == amin).astype(jnp.float32)
    gath = jnp.dot(onehot, src, preferred_element_type=jnp.float32)
    work = work - onehot * jnp.float32(1e9)
    return work, gath


def _stage1_kernel(xt_ref, wq_ref, wk_ref, wv_ref, wbig_ref, out_ref):
    xt = xt_ref[0]
    q = jnp.dot(xt, wq_ref[...].T, preferred_element_type=jnp.float32)
    kk = jnp.dot(xt, wk_ref[...].T, preferred_element_type=jnp.float32)
    v = jnp.dot(xt, wv_ref[...].T, preferred_element_type=jnp.float32)
    logits = jax.lax.dot_general(
        q, kk, (((1,), (1,)), ((), ())),
        preferred_element_type=jnp.float32) / math.sqrt(3.0)
    mx = jnp.max(logits, axis=1, keepdims=True)
    e = jnp.exp(logits - mx)
    attn = e / jnp.sum(e, axis=1, keepdims=True)
    out = jnp.dot(attn, v, preferred_element_type=jnp.float32)
    x2t = xt + out

    colid = jax.lax.broadcasted_iota(jnp.int32, (N, N), 1)
    work = logits
    pieces = []
    for _ in range(K):
        work, gath = _topk_step(work, colid, x2t)
        pieces.append(gath - x2t)
        pieces.append(x2t)
    f = jnp.concatenate(pieces, axis=1)

    acc = jnp.zeros((N, 18 * 64), dtype=jnp.float32)
    for di in range(3):
        fd = f if di == 0 else jnp.concatenate([f[di:], f[:di]], axis=0)
        acc = acc + jnp.dot(fd, wbig_ref[di],
                            preferred_element_type=jnp.float32)
    h = _lrelu(acc)
    m = h[:, 0:64]
    for j in range(1, 18):
        m = jnp.maximum(m, h[:, j * 64:(j + 1) * 64])
    out_ref[0] = m


def _stage2_kernel(x1_ref, wq_ref, wk_ref, wv_ref, wa_ref, wb_ref,
                   s2_ref, b2_ref, w5_ref, s5_ref, b5_ref, out_ref):
    rowid = jax.lax.broadcasted_iota(jnp.int32, (N1P, 64), 0)
    xt = jnp.where(rowid < NV, x1_ref[0], 0.0)
    q = jnp.dot(xt, wq_ref[...].T, preferred_element_type=jnp.float32)
    kk = jnp.dot(xt, wk_ref[...].T, preferred_element_type=jnp.float32)
    v = jnp.dot(xt, wv_ref[...].T, preferred_element_type=jnp.float32)
    colid = jax.lax.broadcasted_iota(jnp.int32, (N1P, N1P), 1)
    logits = jax.lax.dot_general(
        q, kk, (((1,), (1,)), ((), ())),
        preferred_element_type=jnp.float32) * 0.125
    logits = jnp.where(colid < NV, logits, -1e30)
    mx = jnp.max(logits, axis=1, keepdims=True)
    e = jnp.exp(logits - mx)
    attn = e / jnp.sum(e, axis=1, keepdims=True)
    out = jnp.dot(attn, v, preferred_element_type=jnp.float32)
    x2t = xt + out

    base = jnp.dot(x2t, wb_ref[...], preferred_element_type=jnp.float32)
    s2 = s2_ref[...]
    b2 = b2_ref[...]
    work = logits
    x2acc = jnp.full((N1P, 128), -1e30, dtype=jnp.float32)
    for _ in range(K):
        work, gath = _topk_step(work, colid, x2t)
        z = jnp.dot(gath - x2t, wa_ref[...],
                    preferred_element_type=jnp.float32) + base
        x2acc = jnp.maximum(x2acc, _lrelu(z * s2 + b2))
    xcat = jnp.concatenate([xt, x2acc], axis=1)
    h5 = jnp.dot(xcat, w5_ref[...], preferred_element_type=jnp.float32)
    act5 = _lrelu(h5 * s5_ref[...] + b5_ref[...])
    rid5 = jax.lax.broadcasted_iota(jnp.int32, (N1P, 1024), 0)
    act5 = jnp.where(rid5 < NV, act5, -1e30)
    out_ref[0, 0] = jnp.max(act5, axis=0)


def _full(shape):
    nd = len(shape)
    return pl.BlockSpec(shape, lambda b, _n=nd: (0,) * _n)


@jax.jit
def kernel(x, wq1, wk1, wv1, wq2, wk2, wv2, conv1_w, conv2_w, bn2_g, bn2_b,
           conv5_w, bn5_g, bn5_b):
    B = x.shape[0]
    inv = jnp.float32(1.0 / math.sqrt(1.0 + 1e-5))
    xt = jnp.transpose(x, (0, 2, 1))

    wbig = jnp.zeros((3, K * 6, 18 * 64), dtype=jnp.float32)
    rows = []
    for di in range(3):
        acc = jnp.zeros((K, 6, 18, 64), dtype=jnp.float32)
        for dj in range(3):
            eye = jnp.eye(K, 18, k=-dj, dtype=jnp.float32)
            wt = jnp.transpose(conv1_w[:, :, di, dj], (1, 0))
            acc = acc + eye[:, None, :, None] * wt[:, None, :][None]
        rows.append(acc.reshape(K * 6, 18 * 64))
    wbig = jnp.stack(rows, axis=0)

    m_full = pl.pallas_call(
        _stage1_kernel,
        grid=(B,),
        in_specs=[
            pl.BlockSpec((1, N, 3), lambda b: (b, 0, 0)),
            _full((3, 3)), _full((3, 3)), _full((3, 3)),
            _full((3, K * 6, 18 * 64)),
        ],
        out_specs=pl.BlockSpec((1, N, 64), lambda b: (b, 0, 0)),
        out_shape=jax.ShapeDtypeStruct((B, N, 64), jnp.float32),
        compiler_params=pltpu.CompilerParams(
            dimension_semantics=("arbitrary",)),
    )(xt, wq1, wk1, wv1, wbig)

    x1t = m_full.reshape(B, N // 2, 2, 64)[:, :, 0, :]

    w2t = jnp.transpose(conv2_w[:, :, 0, 0], (1, 0))
    wa, wb = w2t[:64], w2t[64:]
    s2 = (inv * bn2_g).reshape(1, 128)
    b2 = bn2_b.reshape(1, 128)
    w5 = jnp.transpose(conv5_w[:, :, 0], (1, 0))
    s5 = (inv * bn5_g).reshape(1, 1024)
    b5 = bn5_b.reshape(1, 1024)

    res = pl.pallas_call(
        _stage2_kernel,
        grid=(B,),
        in_specs=[
            pl.BlockSpec((1, N1P, 64), lambda b: (b, 0, 0)),
            _full((64, 64)), _full((64, 64)), _full((64, 64)),
            _full((64, 128)), _full((64, 128)),
            _full((1, 128)), _full((1, 128)),
            _full((192, 1024)), _full((1, 1024)), _full((1, 1024)),
        ],
        out_specs=pl.BlockSpec((1, 1, 1024), lambda b: (b, 0, 0)),
        out_shape=jax.ShapeDtypeStruct((B, 1, 1024), jnp.float32),
        compiler_params=pltpu.CompilerParams(
            dimension_semantics=("arbitrary",)),
    )(x1t, wq2, wk2, wv2, wa, wb, s2, b2, w5, s5, b5)
    return res.reshape(B, 1024)


# speedup vs baseline: 6.4242x; 1.2873x over previous
import functools
import math

import jax
import jax.numpy as jnp
from jax.experimental import pallas as pl
from jax.experimental.pallas import tpu as pltpu

K = 20
N = 1024
N1P = 512
NV = 511


def _lrelu(t):
    return jnp.where(t >= 0, t, 0.2 * t)


def _topk_step(work, colid, src):
    amin = jnp.argmax(work, axis=1, keepdims=True).astype(jnp.int32)
    onehot = (colid == amin).astype(jnp.float32)
    gath = jnp.dot(onehot, src, preferred_element_type=jnp.float32)
    work = work - onehot * jnp.float32(1e9)
    return work, gath


def _stage1_kernel(xt_ref, wq_ref, wk_ref, wv_ref, wbig_ref, out_ref):
    xt = xt_ref[0]
    q = jnp.dot(xt, wq_ref[...].T, preferred_element_type=jnp.float32)
    kk = jnp.dot(xt, wk_ref[...].T, preferred_element_type=jnp.float32)
    v = jnp.dot(xt, wv_ref[...].T, preferred_element_type=jnp.float32)
    logits = jax.lax.dot_general(
        q, kk, (((1,), (1,)), ((), ())),
        preferred_element_type=jnp.float32) / math.sqrt(3.0)
    mx = jnp.max(logits, axis=1, keepdims=True)
    e = jnp.exp(logits - mx)
    attn = e / jnp.sum(e, axis=1, keepdims=True)
    out = jnp.dot(attn, v, preferred_element_type=jnp.float32)
    x2t = xt + out

    colid = jax.lax.broadcasted_iota(jnp.int32, (N, N), 1)
    work = attn
    pieces = []
    for _ in range(K):
        work, gath = _topk_step(work, colid, x2t)
        pieces.append(gath - x2t)
        pieces.append(x2t)
    f = jnp.concatenate(pieces, axis=1)

    acc = jnp.zeros((N, 18 * 64), dtype=jnp.float32)
    for di in range(3):
        fd = f if di == 0 else jnp.concatenate([f[di:], f[:di]], axis=0)
        acc = acc + jnp.dot(fd, wbig_ref[di],
                            preferred_element_type=jnp.float32)
    h = _lrelu(acc)
    m = h[:, 0:64]
    for j in range(1, 18):
        m = jnp.maximum(m, h[:, j * 64:(j + 1) * 64])
    out_ref[0] = m


def _stage2_kernel(x1_ref, wq_ref, wk_ref, wv_ref, wa_ref, wb_ref,
                   s2_ref, b2_ref, w5_ref, s5_ref, b5_ref, out_ref):
    rowid = jax.lax.broadcasted_iota(jnp.int32, (N1P, 64), 0)
    xt = jnp.where(rowid < NV, x1_ref[0], 0.0)
    q = jnp.dot(xt, wq_ref[...].T, preferred_element_type=jnp.float32)
    kk = jnp.dot(xt, wk_ref[...].T, preferred_element_type=jnp.float32)
    v = jnp.dot(xt, wv_ref[...].T, preferred_element_type=jnp.float32)
    colid = jax.lax.broadcasted_iota(jnp.int32, (N1P, N1P), 1)
    logits = jax.lax.dot_general(
        q, kk, (((1,), (1,)), ((), ())),
        preferred_element_type=jnp.float32) * 0.125
    logits = jnp.where(colid < NV, logits, -1e30)
    mx = jnp.max(logits, axis=1, keepdims=True)
    e = jnp.exp(logits - mx)
    attn = e / jnp.sum(e, axis=1, keepdims=True)
    out = jnp.dot(attn, v, preferred_element_type=jnp.float32)
    x2t = xt + out

    base = jnp.dot(x2t, wb_ref[...], preferred_element_type=jnp.float32)
    s2 = s2_ref[...]
    b2 = b2_ref[...]
    work = attn
    x2acc = jnp.full((N1P, 128), -1e30, dtype=jnp.float32)
    for _ in range(K):
        work, gath = _topk_step(work, colid, x2t)
        z = jnp.dot(gath - x2t, wa_ref[...],
                    preferred_element_type=jnp.float32) + base
        x2acc = jnp.maximum(x2acc, _lrelu(z * s2 + b2))
    xcat = jnp.concatenate([xt, x2acc], axis=1)
    h5 = jnp.dot(xcat, w5_ref[...], preferred_element_type=jnp.float32)
    act5 = _lrelu(h5 * s5_ref[...] + b5_ref[...])
    rid5 = jax.lax.broadcasted_iota(jnp.int32, (N1P, 1024), 0)
    act5 = jnp.where(rid5 < NV, act5, -1e30)
    out_ref[0, 0] = jnp.max(act5, axis=0)


def _full(shape):
    nd = len(shape)
    return pl.BlockSpec(shape, lambda b, _n=nd: (0,) * _n)


@jax.jit
def kernel(x, wq1, wk1, wv1, wq2, wk2, wv2, conv1_w, conv2_w, bn2_g, bn2_b,
           conv5_w, bn5_g, bn5_b):
    B = x.shape[0]
    inv = jnp.float32(1.0 / math.sqrt(1.0 + 1e-5))
    xt = jnp.transpose(x, (0, 2, 1))

    wbig = jnp.zeros((3, K * 6, 18 * 64), dtype=jnp.float32)
    rows = []
    for di in range(3):
        acc = jnp.zeros((K, 6, 18, 64), dtype=jnp.float32)
        for dj in range(3):
            eye = jnp.eye(K, 18, k=-dj, dtype=jnp.float32)
            wt = jnp.transpose(conv1_w[:, :, di, dj], (1, 0))
            acc = acc + eye[:, None, :, None] * wt[:, None, :][None]
        rows.append(acc.reshape(K * 6, 18 * 64))
    wbig = jnp.stack(rows, axis=0)

    m_full = pl.pallas_call(
        _stage1_kernel,
        grid=(B,),
        in_specs=[
            pl.BlockSpec((1, N, 3), lambda b: (b, 0, 0)),
            _full((3, 3)), _full((3, 3)), _full((3, 3)),
            _full((3, K * 6, 18 * 64)),
        ],
        out_specs=pl.BlockSpec((1, N, 64), lambda b: (b, 0, 0)),
        out_shape=jax.ShapeDtypeStruct((B, N, 64), jnp.float32),
        compiler_params=pltpu.CompilerParams(
            dimension_semantics=("parallel",)),
    )(xt, wq1, wk1, wv1, wbig)

    x1t = m_full.reshape(B, N // 2, 2, 64)[:, :, 0, :]

    w2t = jnp.transpose(conv2_w[:, :, 0, 0], (1, 0))
    wa, wb = w2t[:64], w2t[64:]
    s2 = (inv * bn2_g).reshape(1, 128)
    b2 = bn2_b.reshape(1, 128)
    w5 = jnp.transpose(conv5_w[:, :, 0], (1, 0))
    s5 = (inv * bn5_g).reshape(1, 1024)
    b5 = bn5_b.reshape(1, 1024)

    res = pl.pallas_call(
        _stage2_kernel,
        grid=(B,),
        in_specs=[
            pl.BlockSpec((1, N1P, 64), lambda b: (b, 0, 0)),
            _full((64, 64)), _full((64, 64)), _full((64, 64)),
            _full((64, 128)), _full((64, 128)),
            _full((1, 128)), _full((1, 128)),
            _full((192, 1024)), _full((1, 1024)), _full((1, 1024)),
        ],
        out_specs=pl.BlockSpec((1, 1, 1024), lambda b: (b, 0, 0)),
        out_shape=jax.ShapeDtypeStruct((B, 1, 1024), jnp.float32),
        compiler_params=pltpu.CompilerParams(
            dimension_semantics=("parallel",)),
    )(x1t, wq2, wk2, wv2, wa, wb, s2, b2, w5, s5, b5)
    return res.reshape(B, 1024)


# speedup vs baseline: 7.3361x; 1.1419x over previous
import functools
import math

import jax
import jax.numpy as jnp
from jax.experimental import pallas as pl
from jax.experimental.pallas import tpu as pltpu

K = 20
N = 1024
N1P = 512
NV = 511


def _lrelu(t):
    return jnp.where(t >= 0, t, 0.2 * t)


def _topk_step(work, colid, src):
    amin = jnp.argmax(work, axis=1, keepdims=True).astype(jnp.int32)
    eq = colid == amin
    onehot = jnp.where(eq, jnp.float32(1.0), jnp.float32(0.0))
    gath = jnp.dot(onehot, src, preferred_element_type=jnp.float32)
    work = jnp.where(eq, jnp.float32(-1e9), work)
    return work, gath


def _stage1_kernel(xt_ref, wq_ref, wk_ref, wv_ref, wbig_ref, out_ref):
    xt = xt_ref[0]
    q = jnp.dot(xt, wq_ref[...].T, preferred_element_type=jnp.float32)
    kk = jnp.dot(xt, wk_ref[...].T, preferred_element_type=jnp.float32)
    v = jnp.dot(xt, wv_ref[...].T, preferred_element_type=jnp.float32)
    logits = jax.lax.dot_general(
        q, kk, (((1,), (1,)), ((), ())),
        preferred_element_type=jnp.float32) / math.sqrt(3.0)
    mx = jnp.max(logits, axis=1, keepdims=True)
    e = jnp.exp(logits - mx)
    attn = e / jnp.sum(e, axis=1, keepdims=True)
    out = jnp.dot(attn, v, preferred_element_type=jnp.float32)
    x2t = xt + out

    colid = jax.lax.broadcasted_iota(jnp.int32, (N, N), 1)
    work = attn
    pieces = []
    for _ in range(K):
        work, gath = _topk_step(work, colid, x2t)
        pieces.append(gath - x2t)
        pieces.append(x2t)
    f = jnp.concatenate(pieces, axis=1)

    acc = jnp.zeros((N, 18 * 64), dtype=jnp.float32)
    for di in range(3):
        fd = f if di == 0 else jnp.concatenate([f[di:], f[:di]], axis=0)
        acc = acc + jnp.dot(fd, wbig_ref[di],
                            preferred_element_type=jnp.float32)
    h = _lrelu(acc)
    m = h[:, 0:64]
    for j in range(1, 18):
        m = jnp.maximum(m, h[:, j * 64:(j + 1) * 64])
    out_ref[0] = m


def _stage2_kernel(x1_ref, wq_ref, wk_ref, wv_ref, wa_ref, wb_ref,
                   s2_ref, b2_ref, w5_ref, s5_ref, b5_ref, out_ref):
    rowid = jax.lax.broadcasted_iota(jnp.int32, (N1P, 64), 0)
    xt = jnp.where(rowid < NV, x1_ref[0], 0.0)
    q = jnp.dot(xt, wq_ref[...].T, preferred_element_type=jnp.float32)
    kk = jnp.dot(xt, wk_ref[...].T, preferred_element_type=jnp.float32)
    v = jnp.dot(xt, wv_ref[...].T, preferred_element_type=jnp.float32)
    colid = jax.lax.broadcasted_iota(jnp.int32, (N1P, N1P), 1)
    logits = jax.lax.dot_general(
        q, kk, (((1,), (1,)), ((), ())),
        preferred_element_type=jnp.float32) * 0.125
    logits = jnp.where(colid < NV, logits, -1e30)
    mx = jnp.max(logits, axis=1, keepdims=True)
    e = jnp.exp(logits - mx)
    attn = e / jnp.sum(e, axis=1, keepdims=True)
    out = jnp.dot(attn, v, preferred_element_type=jnp.float32)
    x2t = xt + out

    base = jnp.dot(x2t, wb_ref[...], preferred_element_type=jnp.float32)
    s2 = s2_ref[...]
    b2 = b2_ref[...]
    work = attn
    x2acc = jnp.full((N1P, 128), -1e30, dtype=jnp.float32)
    for _ in range(K):
        work, gath = _topk_step(work, colid, x2t)
        z = jnp.dot(gath - x2t, wa_ref[...],
                    preferred_element_type=jnp.float32) + base
        x2acc = jnp.maximum(x2acc, _lrelu(z * s2 + b2))
    xcat = jnp.concatenate([xt, x2acc], axis=1)
    h5 = jnp.dot(xcat, w5_ref[...], preferred_element_type=jnp.float32)
    act5 = _lrelu(h5 * s5_ref[...] + b5_ref[...])
    rid5 = jax.lax.broadcasted_iota(jnp.int32, (N1P, 1024), 0)
    act5 = jnp.where(rid5 < NV, act5, -1e30)
    out_ref[0, 0] = jnp.max(act5, axis=0)


def _full(shape):
    nd = len(shape)
    return pl.BlockSpec(shape, lambda b, _n=nd: (0,) * _n)


@jax.jit
def kernel(x, wq1, wk1, wv1, wq2, wk2, wv2, conv1_w, conv2_w, bn2_g, bn2_b,
           conv5_w, bn5_g, bn5_b):
    B = x.shape[0]
    inv = jnp.float32(1.0 / math.sqrt(1.0 + 1e-5))
    xt = jnp.transpose(x, (0, 2, 1))

    wbig = jnp.zeros((3, K * 6, 18 * 64), dtype=jnp.float32)
    rows = []
    for di in range(3):
        acc = jnp.zeros((K, 6, 18, 64), dtype=jnp.float32)
        for dj in range(3):
            eye = jnp.eye(K, 18, k=-dj, dtype=jnp.float32)
            wt = jnp.transpose(conv1_w[:, :, di, dj], (1, 0))
            acc = acc + eye[:, None, :, None] * wt[:, None, :][None]
        rows.append(acc.reshape(K * 6, 18 * 64))
    wbig = jnp.stack(rows, axis=0)

    m_full = pl.pallas_call(
        _stage1_kernel,
        grid=(B,),
        in_specs=[
            pl.BlockSpec((1, N, 3), lambda b: (b, 0, 0)),
            _full((3, 3)), _full((3, 3)), _full((3, 3)),
            _full((3, K * 6, 18 * 64)),
        ],
        out_specs=pl.BlockSpec((1, N, 64), lambda b: (b, 0, 0)),
        out_shape=jax.ShapeDtypeStruct((B, N, 64), jnp.float32),
        compiler_params=pltpu.CompilerParams(
            dimension_semantics=("parallel",)),
    )(xt, wq1, wk1, wv1, wbig)

    x1t = m_full.reshape(B, N // 2, 2, 64)[:, :, 0, :]

    w2t = jnp.transpose(conv2_w[:, :, 0, 0], (1, 0))
    wa, wb = w2t[:64], w2t[64:]
    s2 = (inv * bn2_g).reshape(1, 128)
    b2 = bn2_b.reshape(1, 128)
    w5 = jnp.transpose(conv5_w[:, :, 0], (1, 0))
    s5 = (inv * bn5_g).reshape(1, 1024)
    b5 = bn5_b.reshape(1, 1024)

    res = pl.pallas_call(
        _stage2_kernel,
        grid=(B,),
        in_specs=[
            pl.BlockSpec((1, N1P, 64), lambda b: (b, 0, 0)),
            _full((64, 64)), _full((64, 64)), _full((64, 64)),
            _full((64, 128)), _full((64, 128)),
            _full((1, 128)), _full((1, 128)),
            _full((192, 1024)), _full((1, 1024)), _full((1, 1024)),
        ],
        out_specs=pl.BlockSpec((1, 1, 1024), lambda b: (b, 0, 0)),
        out_shape=jax.ShapeDtypeStruct((B, 1, 1024), jnp.float32),
        compiler_params=pltpu.CompilerParams(
            dimension_semantics=("parallel",)),
    )(x1t, wq2, wk2, wv2, wa, wb, s2, b2, w5, s5, b5)
    return res.reshape(B, 1024)
